# Initial kernel scaffold; baseline (speedup 1.0000x reference)
import functools

import jax
import jax.numpy as jnp
from jax import lax
from jax.experimental import pallas as pl
from jax.experimental.pallas import tpu as pltpu
from jax.experimental.pallas import tpu_sc as plsc

NSAMPLE = 64
L = 16
NUM_TILES = 32
B, N, NQ, C = 4, 8192, 1024, 128
Q_PER_TILE = NQ * B // NUM_TILES
TILES_PER_BATCH = NUM_TILES // B
C_PER_TILE = C // TILES_PER_BATCH
NKV = N // L
QCHUNK = 512


def _mesh():
    return plsc.VectorSubcoreMesh(core_axis_name="c", subcore_axis_name="s")


def _wid():
    return lax.axis_index("s") * 2 + lax.axis_index("c")


def _select_body(coords_hbm, query_hbm, idx_hbm, mask_hbm, gxyz_hbm,
                 coords_v, q_v, idx_v, mask_v, gxyz_v):
    wid = _wid()
    b = wid // TILES_PER_BATCH
    qbase = (wid % TILES_PER_BATCH) * Q_PER_TILE

    pltpu.sync_copy(coords_hbm.at[b], coords_v)
    pltpu.sync_copy(query_hbm.at[b, :, pl.ds(qbase, Q_PER_TILE)], q_v)

    lane = jnp.arange(L, dtype=jnp.int32)
    zeros_i = jnp.zeros((L,), jnp.int32)

    def qloop(q, _):
        qsplat = jnp.full((L,), q, jnp.int32)

        def qval(d):
            return plsc.load_gather(q_v, [jnp.full((L,), d, jnp.int32), qsplat])

        cx, cy, cz = qval(0), qval(1), qval(2)
        hx, hy, hz = 0.5 * qval(3), 0.5 * qval(4), 0.5 * qval(5)

        for j in range(NSAMPLE // L):
            idx_v[q, pl.ds(j * L, L)] = zeros_i

        def cond(carry):
            i, cnt = carry
            return jnp.logical_and(i < NKV, cnt < NSAMPLE)

        def body(carry):
            i, cnt = carry
            base = i * L
            xv = coords_v[0, pl.ds(base, L)]
            yv = coords_v[1, pl.ds(base, L)]
            zv = coords_v[2, pl.ds(base, L)]
            inside = jnp.logical_and(
                jnp.logical_and(jnp.abs(xv - cx) <= hx, jnp.abs(yv - cy) <= hy),
                jnp.abs(zv - cz) <= hz)
            ii = inside.astype(jnp.int32)
            pos = cnt + plsc.cumsum(ii) - 1
            m = jnp.logical_and(inside, pos < NSAMPLE)
            posc = jnp.minimum(pos, NSAMPLE - 1)
            keyidx = base + lane
            plsc.store_scatter(idx_v, [qsplat, posc], keyidx, mask=m)
            return i + 1, cnt + jnp.sum(ii)

        _, cnt = lax.while_loop(cond, body, (jnp.int32(0), jnp.int32(0)))
        cntv = jnp.full((L,), jnp.minimum(cnt, NSAMPLE), jnp.int32)

        for j in range(NSAMPLE // L):
            s_ids = j * L + lane
            idxv = idx_v[q, pl.ds(j * L, L)]
            invalid = s_ids >= cntv
            if j == 0:
                invalid = jnp.logical_and(invalid, s_ids != 0)
            mask_v[q, pl.ds(j * L, L)] = invalid.astype(jnp.int32)
            gxyz_v[0, q, pl.ds(j * L, L)] = plsc.load_gather(
                coords_v, [jnp.full((L,), 0, jnp.int32), idxv]) - cx
            gxyz_v[1, q, pl.ds(j * L, L)] = plsc.load_gather(
                coords_v, [jnp.full((L,), 1, jnp.int32), idxv]) - cy
            gxyz_v[2, q, pl.ds(j * L, L)] = plsc.load_gather(
                coords_v, [jnp.full((L,), 2, jnp.int32), idxv]) - cz
        return 0

    lax.fori_loop(0, Q_PER_TILE, qloop, 0)

    pltpu.sync_copy(idx_v, idx_hbm.at[b, pl.ds(qbase, Q_PER_TILE)])
    pltpu.sync_copy(mask_v, mask_hbm.at[b, pl.ds(qbase, Q_PER_TILE)])
    pltpu.sync_copy(gxyz_v, gxyz_hbm.at[b, :, pl.ds(qbase, Q_PER_TILE)])


def _gather_body(feat_hbm, idx_hbm, out_hbm, idx_v, row_v, out_v):
    wid = _wid()
    b = wid // TILES_PER_BATCH
    cbase = (wid % TILES_PER_BATCH) * C_PER_TILE

    for qc in range(NQ // QCHUNK):
        pltpu.sync_copy(idx_hbm.at[b, pl.ds(qc * QCHUNK, QCHUNK)], idx_v)
        for c in range(C_PER_TILE):
            pltpu.sync_copy(feat_hbm.at[b, cbase + c], row_v)

            def rloop(r, _):
                for j in range(NSAMPLE // L):
                    idxv = idx_v[r, pl.ds(j * L, L)]
                    out_v[r, pl.ds(j * L, L)] = plsc.load_gather(row_v, [idxv])
                return 0

            lax.fori_loop(0, QCHUNK, rloop, 0)
            pltpu.sync_copy(
                out_v, out_hbm.at[b, cbase + c, pl.ds(qc * QCHUNK, QCHUNK)])


@jax.jit
def _run(coords, q_t, key_features):
    select = pl.kernel(
        _select_body,
        out_type=[
            jax.ShapeDtypeStruct((B, NQ, NSAMPLE), jnp.int32),
            jax.ShapeDtypeStruct((B, NQ, NSAMPLE), jnp.int32),
            jax.ShapeDtypeStruct((B, 3, NQ, NSAMPLE), jnp.float32),
        ],
        mesh=_mesh(),
        scratch_types=[
            pltpu.VMEM((3, N), jnp.float32),
            pltpu.VMEM((6, Q_PER_TILE), jnp.float32),
            pltpu.VMEM((Q_PER_TILE, NSAMPLE), jnp.int32),
            pltpu.VMEM((Q_PER_TILE, NSAMPLE), jnp.int32),
            pltpu.VMEM((3, Q_PER_TILE, NSAMPLE), jnp.float32),
        ],
    )
    idx, mask_i, gxyz = select(coords, q_t)

    gather = pl.kernel(
        _gather_body,
        out_type=jax.ShapeDtypeStruct((B, C, NQ, NSAMPLE), jnp.float32),
        mesh=_mesh(),
        scratch_types=[
            pltpu.VMEM((QCHUNK, NSAMPLE), jnp.int32),
            pltpu.VMEM((N,), jnp.float32),
            pltpu.VMEM((QCHUNK, NSAMPLE), jnp.float32),
        ],
    )
    gfeat = gather(key_features, idx)
    return gxyz, gfeat, mask_i.astype(bool)


def kernel(key_xyz, key_features, query_xyz):
    coords = jnp.transpose(key_xyz, (0, 2, 1))
    q_t = jnp.transpose(query_xyz, (0, 2, 1))
    return _run(coords, q_t, key_features)


# speedup vs baseline: 12.9231x; 12.9231x over previous
import jax
import jax.numpy as jnp
from jax import lax
from jax.experimental import pallas as pl
from jax.experimental.pallas import tpu as pltpu
from jax.experimental.pallas import tpu_sc as plsc

NSAMPLE = 64
L = 16
NUM_TILES = 32
B, N, NQ, C = 4, 8192, 1024, 128
Q_PER_TILE = NQ * B // NUM_TILES
TILES_PER_BATCH = NUM_TILES // B
C_PER_TILE = C // TILES_PER_BATCH
NKV = N // L
QCHUNK = 512


def _mesh():
    return plsc.VectorSubcoreMesh(core_axis_name="c", subcore_axis_name="s")


def _params():
    return pltpu.CompilerParams(needs_layout_passes=False)


def _wid():
    return lax.axis_index("s") * 2 + lax.axis_index("c")


def _select_body(coords_hbm, query_hbm, idx_hbm, mask_hbm, gxyz_hbm,
                 xs_v, ys_v, zs_v, q_v, idx_v, mask_v, gx_v, gy_v, gz_v):
    wid = _wid()
    b = wid // TILES_PER_BATCH
    qbase = (wid % TILES_PER_BATCH) * Q_PER_TILE

    pltpu.sync_copy(coords_hbm.at[pl.ds((b * 3 + 0) * N, N)], xs_v)
    pltpu.sync_copy(coords_hbm.at[pl.ds((b * 3 + 1) * N, N)], ys_v)
    pltpu.sync_copy(coords_hbm.at[pl.ds((b * 3 + 2) * N, N)], zs_v)
    for d in range(6):
        pltpu.sync_copy(
            query_hbm.at[pl.ds((b * 6 + d) * NQ + qbase, Q_PER_TILE)],
            q_v.at[pl.ds(d * Q_PER_TILE, Q_PER_TILE)])

    lane = jnp.arange(L, dtype=jnp.int32)
    zeros_i = jnp.zeros((L,), jnp.int32)

    def qloop(q, _):
        qsplat = jnp.full((L,), q, jnp.int32)

        def qval(d):
            return plsc.load_gather(q_v, [d * Q_PER_TILE + qsplat])

        cx, cy, cz = qval(0), qval(1), qval(2)
        hx, hy, hz = 0.5 * qval(3), 0.5 * qval(4), 0.5 * qval(5)
        obase = q * NSAMPLE

        for j in range(NSAMPLE // L):
            idx_v[pl.ds(obase + j * L, L)] = zeros_i

        def cond(carry):
            i, cnt = carry
            return jnp.logical_and(i < NKV, cnt < NSAMPLE)

        def body(carry):
            i, cnt = carry
            base = i * L
            xv = xs_v[pl.ds(base, L)]
            yv = ys_v[pl.ds(base, L)]
            zv = zs_v[pl.ds(base, L)]
            inside = jnp.logical_and(
                jnp.logical_and(jnp.abs(xv - cx) <= hx, jnp.abs(yv - cy) <= hy),
                jnp.abs(zv - cz) <= hz)
            ii = inside.astype(jnp.int32)
            pos = cnt + plsc.cumsum(ii) - 1
            m = jnp.logical_and(inside, pos < NSAMPLE)
            posc = obase + jnp.minimum(pos, NSAMPLE - 1)
            keyidx = base + lane
            plsc.store_scatter(idx_v, [posc], keyidx, mask=m)
            return i + 1, cnt + jnp.sum(ii)

        _, cnt = lax.while_loop(cond, body, (jnp.int32(0), jnp.int32(0)))
        cntv = jnp.full((L,), jnp.minimum(cnt, NSAMPLE), jnp.int32)

        for j in range(NSAMPLE // L):
            s_ids = j * L + lane
            idxv = idx_v[pl.ds(obase + j * L, L)]
            invalid = s_ids >= cntv
            if j == 0:
                invalid = jnp.logical_and(invalid, s_ids != 0)
            mask_v[pl.ds(obase + j * L, L)] = invalid.astype(jnp.int32)
            gx_v[pl.ds(obase + j * L, L)] = plsc.load_gather(xs_v, [idxv]) - cx
            gy_v[pl.ds(obase + j * L, L)] = plsc.load_gather(ys_v, [idxv]) - cy
            gz_v[pl.ds(obase + j * L, L)] = plsc.load_gather(zs_v, [idxv]) - cz
        return 0

    lax.fori_loop(0, Q_PER_TILE, qloop, 0)

    flen = Q_PER_TILE * NSAMPLE
    fbase = (b * NQ + qbase) * NSAMPLE
    pltpu.sync_copy(idx_v, idx_hbm.at[pl.ds(fbase, flen)])
    pltpu.sync_copy(mask_v, mask_hbm.at[pl.ds(fbase, flen)])
    gbase = ((b * 3 + 0) * NQ + qbase) * NSAMPLE
    pltpu.sync_copy(gx_v, gxyz_hbm.at[pl.ds(gbase, flen)])
    pltpu.sync_copy(gy_v, gxyz_hbm.at[pl.ds(gbase + NQ * NSAMPLE, flen)])
    pltpu.sync_copy(gz_v, gxyz_hbm.at[pl.ds(gbase + 2 * NQ * NSAMPLE, flen)])


def _gather_body(feat_hbm, idx_hbm, out_hbm, idx_v, row_v, out_v):
    wid = _wid()
    b = wid // TILES_PER_BATCH
    cbase = (wid % TILES_PER_BATCH) * C_PER_TILE
    clen = QCHUNK * NSAMPLE

    for qc in range(NQ // QCHUNK):
        pltpu.sync_copy(
            idx_hbm.at[pl.ds(b * NQ * NSAMPLE + qc * clen, clen)], idx_v)
        for c in range(C_PER_TILE):
            pltpu.sync_copy(feat_hbm.at[pl.ds((b * C + cbase + c) * N, N)],
                            row_v)

            def rloop(r, _):
                rb = r * NSAMPLE
                for j in range(NSAMPLE // L):
                    idxv = idx_v[pl.ds(rb + j * L, L)]
                    out_v[pl.ds(rb + j * L, L)] = plsc.load_gather(row_v, [idxv])
                return 0

            lax.fori_loop(0, QCHUNK, rloop, 0)
            obase = ((b * C + cbase + c) * NQ * NSAMPLE) + qc * clen
            pltpu.sync_copy(out_v, out_hbm.at[pl.ds(obase, clen)])


@jax.jit
def _run(coords, q_t, key_features):
    select = pl.kernel(
        _select_body,
        out_type=[
            jax.ShapeDtypeStruct((B * NQ * NSAMPLE,), jnp.int32),
            jax.ShapeDtypeStruct((B * NQ * NSAMPLE,), jnp.int32),
            jax.ShapeDtypeStruct((B * 3 * NQ * NSAMPLE,), jnp.float32),
        ],
        mesh=_mesh(),
        compiler_params=_params(),
        scratch_types=[
            pltpu.VMEM((N,), jnp.float32),
            pltpu.VMEM((N,), jnp.float32),
            pltpu.VMEM((N,), jnp.float32),
            pltpu.VMEM((6 * Q_PER_TILE,), jnp.float32),
            pltpu.VMEM((Q_PER_TILE * NSAMPLE,), jnp.int32),
            pltpu.VMEM((Q_PER_TILE * NSAMPLE,), jnp.int32),
            pltpu.VMEM((Q_PER_TILE * NSAMPLE,), jnp.float32),
            pltpu.VMEM((Q_PER_TILE * NSAMPLE,), jnp.float32),
            pltpu.VMEM((Q_PER_TILE * NSAMPLE,), jnp.float32),
        ],
    )
    idx, mask_i, gxyz = select(coords, q_t)

    gather = pl.kernel(
        _gather_body,
        out_type=jax.ShapeDtypeStruct((B * C * NQ * NSAMPLE,), jnp.float32),
        mesh=_mesh(),
        compiler_params=_params(),
        scratch_types=[
            pltpu.VMEM((QCHUNK * NSAMPLE,), jnp.int32),
            pltpu.VMEM((N,), jnp.float32),
            pltpu.VMEM((QCHUNK * NSAMPLE,), jnp.float32),
        ],
    )
    gfeat = gather(key_features, idx)

    gxyz = gxyz.reshape(B, 3, NQ, NSAMPLE)
    gfeat = gfeat.reshape(B, C, NQ, NSAMPLE)
    mask = mask_i.reshape(B, NQ, NSAMPLE).astype(bool)
    return gxyz, gfeat, mask


def kernel(key_xyz, key_features, query_xyz):
    coords = jnp.transpose(key_xyz, (0, 2, 1)).reshape(-1)
    q_t = jnp.transpose(query_xyz, (0, 2, 1)).reshape(-1)
    return _run(coords, q_t, key_features.reshape(-1))


# speedup vs baseline: 20.0949x; 1.5550x over previous
import jax
import jax.numpy as jnp
from jax import lax
from jax.experimental import pallas as pl
from jax.experimental.pallas import tpu as pltpu
from jax.experimental.pallas import tpu_sc as plsc

NSAMPLE = 64
L = 16
NUM_TILES = 32
B, N, NQ, C = 4, 8192, 1024, 128
Q_PER_TILE = NQ * B // NUM_TILES
TILES_PER_BATCH = NUM_TILES // B
C_PER_TILE = C // TILES_PER_BATCH
NKV = N // L
U = 4
QCHUNK = 512
CLEN = QCHUNK * NSAMPLE


def _mesh():
    return plsc.VectorSubcoreMesh(core_axis_name="c", subcore_axis_name="s")


def _params():
    return pltpu.CompilerParams(needs_layout_passes=False)


def _wid():
    return lax.axis_index("s") * 2 + lax.axis_index("c")


def _select_body(coords_hbm, query_hbm, idx_hbm, mask_hbm, gxyz_hbm,
                 cint_v, xs_v, ys_v, zs_v, q_v, idx_v, mask_v,
                 gx_v, gy_v, gz_v):
    wid = _wid()
    b = wid // TILES_PER_BATCH
    qbase = (wid % TILES_PER_BATCH) * Q_PER_TILE

    pltpu.sync_copy(coords_hbm.at[pl.ds(b * N * 3, N * 3)], cint_v)
    pltpu.sync_copy(
        query_hbm.at[pl.ds((b * NQ + qbase) * 6, Q_PER_TILE * 6)], q_v)

    lane = jnp.arange(L, dtype=jnp.int32)
    zeros_i = jnp.zeros((L,), jnp.int32)

    def dloop(i, _):
        kb = i * L
        idx3 = (kb + lane) * 3
        xs_v[pl.ds(kb, L)] = plsc.load_gather(cint_v, [idx3])
        ys_v[pl.ds(kb, L)] = plsc.load_gather(cint_v, [idx3 + 1])
        zs_v[pl.ds(kb, L)] = plsc.load_gather(cint_v, [idx3 + 2])
        return 0

    lax.fori_loop(0, NKV, dloop, 0)

    def qloop(q, _):
        qsplat = jnp.full((L,), q, jnp.int32)
        q6 = qsplat * 6

        def qval(d):
            return plsc.load_gather(q_v, [q6 + d])

        cx, cy, cz = qval(0), qval(1), qval(2)
        hx, hy, hz = 0.5 * qval(3), 0.5 * qval(4), 0.5 * qval(5)
        obase = q * NSAMPLE

        for j in range(NSAMPLE // L):
            idx_v[pl.ds(obase + j * L, L)] = zeros_i

        def cond(carry):
            i, cnt = carry
            return jnp.logical_and(i < NKV, cnt < NSAMPLE)

        def body(carry):
            i, cnt = carry
            kb = i * L
            insides = []
            pcs = []
            for u in range(U):
                xv = xs_v[pl.ds(kb + u * L, L)]
                yv = ys_v[pl.ds(kb + u * L, L)]
                zv = zs_v[pl.ds(kb + u * L, L)]
                inside = jnp.logical_and(
                    jnp.logical_and(jnp.abs(xv - cx) <= hx,
                                    jnp.abs(yv - cy) <= hy),
                    jnp.abs(zv - cz) <= hz)
                insides.append(inside)
                pcs.append(plsc.all_reduce_population_count(inside))
            tot_v = pcs[0] + pcs[1] + pcs[2] + pcs[3]
            tot = tot_v[0]

            @pl.when(tot > 0)
            def _():
                off = obase + cnt
                for u in range(U):
                    plsc.store_compressed(idx_v.at[pl.ds(off, L)],
                                          kb + u * L + lane, mask=insides[u])
                    if u + 1 < U:
                        off = off + pcs[u][0]

            return i + U, cnt + tot

        _, cnt = lax.while_loop(cond, body, (jnp.int32(0), jnp.int32(0)))
        cntv = jnp.full((L,), jnp.minimum(cnt, NSAMPLE), jnp.int32)

        for j in range(NSAMPLE // L):
            s_ids = j * L + lane
            idxv = idx_v[pl.ds(obase + j * L, L)]
            invalid = s_ids >= cntv
            if j == 0:
                invalid = jnp.logical_and(invalid, s_ids != 0)
            mask_v[pl.ds(obase + j * L, L)] = invalid.astype(jnp.int32)
            gx_v[pl.ds(obase + j * L, L)] = plsc.load_gather(xs_v, [idxv]) - cx
            gy_v[pl.ds(obase + j * L, L)] = plsc.load_gather(ys_v, [idxv]) - cy
            gz_v[pl.ds(obase + j * L, L)] = plsc.load_gather(zs_v, [idxv]) - cz
        return 0

    lax.fori_loop(0, Q_PER_TILE, qloop, 0)

    flen = Q_PER_TILE * NSAMPLE
    fbase = (b * NQ + qbase) * NSAMPLE
    pltpu.sync_copy(idx_v.at[pl.ds(0, flen)], idx_hbm.at[pl.ds(fbase, flen)])
    pltpu.sync_copy(mask_v, mask_hbm.at[pl.ds(fbase, flen)])
    gbase = ((b * 3 + 0) * NQ + qbase) * NSAMPLE
    pltpu.sync_copy(gx_v, gxyz_hbm.at[pl.ds(gbase, flen)])
    pltpu.sync_copy(gy_v, gxyz_hbm.at[pl.ds(gbase + NQ * NSAMPLE, flen)])
    pltpu.sync_copy(gz_v, gxyz_hbm.at[pl.ds(gbase + 2 * NQ * NSAMPLE, flen)])


def _gather_body(feat_hbm, idx_hbm, out_hbm, idx_v,
                 row0_v, row1_v, out0_v, out1_v, rsem, osem):
    wid = _wid()
    b = wid // TILES_PER_BATCH
    cbase = (wid % TILES_PER_BATCH) * C_PER_TILE
    rows = [row0_v, row1_v]
    outs = [out0_v, out1_v]

    def _row_copy(c, buf):
        src = feat_hbm.at[pl.ds((b * C + cbase + c) * N, N)]
        return pltpu.async_copy(src, buf, rsem)

    def _out_copy(c, qc, buf):
        obase = ((b * C + cbase + c) * NQ + qc * QCHUNK) * NSAMPLE
        return pltpu.async_copy(buf, out_hbm.at[pl.ds(obase, CLEN)], osem)

    for qc in range(NQ // QCHUNK):
        pltpu.sync_copy(
            idx_hbm.at[pl.ds(b * NQ * NSAMPLE + qc * CLEN, CLEN)], idx_v)
        rd = {0: _row_copy(0, rows[0])}
        od = {}
        for c in range(C_PER_TILE):
            rd[c].wait()
            if c + 1 < C_PER_TILE:
                rd[c + 1] = _row_copy(c + 1, rows[(c + 1) % 2])
            if c - 2 in od:
                od[c - 2].wait()
            row_buf = rows[c % 2]
            out_buf = outs[c % 2]

            def rloop(r4, _):
                rb = r4 * (4 * NSAMPLE)
                for t in range(16):
                    o = rb + t * L
                    idxv = idx_v[pl.ds(o, L)]
                    out_buf[pl.ds(o, L)] = plsc.load_gather(row_buf, [idxv])
                return 0

            lax.fori_loop(0, QCHUNK // 4, rloop, 0)
            od[c] = _out_copy(c, qc, out_buf)
        od[C_PER_TILE - 2].wait()
        od[C_PER_TILE - 1].wait()


@jax.jit
def _run(coords, q_flat, key_features):
    select = pl.kernel(
        _select_body,
        out_type=[
            jax.ShapeDtypeStruct((B * NQ * NSAMPLE,), jnp.int32),
            jax.ShapeDtypeStruct((B * NQ * NSAMPLE,), jnp.int32),
            jax.ShapeDtypeStruct((B * 3 * NQ * NSAMPLE,), jnp.float32),
        ],
        mesh=_mesh(),
        compiler_params=_params(),
        scratch_types=[
            pltpu.VMEM((N * 3,), jnp.float32),
            pltpu.VMEM((N,), jnp.float32),
            pltpu.VMEM((N,), jnp.float32),
            pltpu.VMEM((N,), jnp.float32),
            pltpu.VMEM((Q_PER_TILE * 6,), jnp.float32),
            pltpu.VMEM((Q_PER_TILE * NSAMPLE + NSAMPLE,), jnp.int32),
            pltpu.VMEM((Q_PER_TILE * NSAMPLE,), jnp.int32),
            pltpu.VMEM((Q_PER_TILE * NSAMPLE,), jnp.float32),
            pltpu.VMEM((Q_PER_TILE * NSAMPLE,), jnp.float32),
            pltpu.VMEM((Q_PER_TILE * NSAMPLE,), jnp.float32),
        ],
    )
    idx, mask_i, gxyz = select(coords, q_flat)

    gather = pl.kernel(
        _gather_body,
        out_type=jax.ShapeDtypeStruct((B * C * NQ * NSAMPLE,), jnp.float32),
        mesh=_mesh(),
        compiler_params=_params(),
        scratch_types=[
            pltpu.VMEM((CLEN,), jnp.int32),
            pltpu.VMEM((N,), jnp.float32),
            pltpu.VMEM((N,), jnp.float32),
            pltpu.VMEM((CLEN,), jnp.float32),
            pltpu.VMEM((CLEN,), jnp.float32),
            pltpu.SemaphoreType.DMA,
            pltpu.SemaphoreType.DMA,
        ],
    )
    gfeat = gather(key_features, idx)

    gxyz = gxyz.reshape(B, 3, NQ, NSAMPLE)
    gfeat = gfeat.reshape(B, C, NQ, NSAMPLE)
    mask = mask_i.reshape(B, NQ, NSAMPLE).astype(bool)
    return gxyz, gfeat, mask


def kernel(key_xyz, key_features, query_xyz):
    return _run(key_xyz.reshape(-1), query_xyz.reshape(-1),
                key_features.reshape(-1))


# speedup vs baseline: 25.2665x; 1.2574x over previous
import jax
import jax.numpy as jnp
from jax import lax
from jax.experimental import pallas as pl
from jax.experimental.pallas import tpu as pltpu
from jax.experimental.pallas import tpu_sc as plsc

NSAMPLE = 64
L = 16
NUM_TILES = 32
B, N, NQ, C = 4, 8192, 1024, 128
Q_PER_TILE = NQ * B // NUM_TILES
TILES_PER_BATCH = NUM_TILES // B
C_PER_TILE = C // TILES_PER_BATCH
NKV = N // L
U = 4
QCHUNK = 512
CLEN = QCHUNK * NSAMPLE


def _mesh():
    return plsc.VectorSubcoreMesh(core_axis_name="c", subcore_axis_name="s")


def _params():
    return pltpu.CompilerParams(needs_layout_passes=False)


def _wid():
    return lax.axis_index("s") * 2 + lax.axis_index("c")


def _select_body(coords_hbm, query_hbm, idx_hbm, mask_hbm, gxyz_hbm,
                 cint_v, xs_v, ys_v, zs_v, q_v, idx_v, mask_v,
                 gx_v, gy_v, gz_v):
    wid = _wid()
    b = wid // TILES_PER_BATCH
    qbase = (wid % TILES_PER_BATCH) * Q_PER_TILE

    pltpu.sync_copy(coords_hbm.at[pl.ds(b * N * 3, N * 3)], cint_v)
    pltpu.sync_copy(
        query_hbm.at[pl.ds((b * NQ + qbase) * 6, Q_PER_TILE * 6)], q_v)

    lane = jnp.arange(L, dtype=jnp.int32)
    zeros_i = jnp.zeros((L,), jnp.int32)

    @plsc.parallel_loop(0, N, L, unroll=8)
    def dloop(kb):
        idx3 = (kb + lane) * 3
        xs_v[pl.ds(kb, L)] = plsc.load_gather(cint_v, [idx3])
        ys_v[pl.ds(kb, L)] = plsc.load_gather(cint_v, [idx3 + 1])
        zs_v[pl.ds(kb, L)] = plsc.load_gather(cint_v, [idx3 + 2])

    def qloop(q, _):
        qsplat = jnp.full((L,), q, jnp.int32)
        q6 = qsplat * 6

        def qval(d):
            return plsc.load_gather(q_v, [q6 + d])

        cx, cy, cz = qval(0), qval(1), qval(2)
        hx, hy, hz = 0.5 * qval(3), 0.5 * qval(4), 0.5 * qval(5)
        obase = q * NSAMPLE

        for j in range(NSAMPLE // L):
            idx_v[pl.ds(obase + j * L, L)] = zeros_i

        def cond(carry):
            i, cnt = carry
            return jnp.logical_and(i < NKV, cnt < NSAMPLE)

        def body(carry):
            i, cnt = carry
            kb = i * L
            insides = []
            pcs = []
            for u in range(U):
                xv = xs_v[pl.ds(kb + u * L, L)]
                yv = ys_v[pl.ds(kb + u * L, L)]
                zv = zs_v[pl.ds(kb + u * L, L)]
                inside = jnp.logical_and(
                    jnp.logical_and(jnp.abs(xv - cx) <= hx,
                                    jnp.abs(yv - cy) <= hy),
                    jnp.abs(zv - cz) <= hz)
                insides.append(inside)
                pcs.append(plsc.all_reduce_population_count(inside))
            tot_v = pcs[0] + pcs[1] + pcs[2] + pcs[3]
            tot = tot_v[0]

            @pl.when(tot > 0)
            def _():
                off = obase + cnt
                for u in range(U):
                    plsc.store_compressed(idx_v.at[pl.ds(off, L)],
                                          kb + u * L + lane, mask=insides[u])
                    if u + 1 < U:
                        off = off + pcs[u][0]

            return i + U, cnt + tot

        _, cnt = lax.while_loop(cond, body, (jnp.int32(0), jnp.int32(0)))
        cntv = jnp.full((L,), jnp.minimum(cnt, NSAMPLE), jnp.int32)

        for j in range(NSAMPLE // L):
            s_ids = j * L + lane
            idxv = idx_v[pl.ds(obase + j * L, L)]
            invalid = s_ids >= cntv
            if j == 0:
                invalid = jnp.logical_and(invalid, s_ids != 0)
            mask_v[pl.ds(obase + j * L, L)] = invalid.astype(jnp.int32)
            gx_v[pl.ds(obase + j * L, L)] = plsc.load_gather(xs_v, [idxv]) - cx
            gy_v[pl.ds(obase + j * L, L)] = plsc.load_gather(ys_v, [idxv]) - cy
            gz_v[pl.ds(obase + j * L, L)] = plsc.load_gather(zs_v, [idxv]) - cz
        return 0

    lax.fori_loop(0, Q_PER_TILE, qloop, 0)

    flen = Q_PER_TILE * NSAMPLE
    fbase = (b * NQ + qbase) * NSAMPLE
    pltpu.sync_copy(idx_v.at[pl.ds(0, flen)], idx_hbm.at[pl.ds(fbase, flen)])
    pltpu.sync_copy(mask_v, mask_hbm.at[pl.ds(fbase, flen)])
    gbase = ((b * 3 + 0) * NQ + qbase) * NSAMPLE
    pltpu.sync_copy(gx_v, gxyz_hbm.at[pl.ds(gbase, flen)])
    pltpu.sync_copy(gy_v, gxyz_hbm.at[pl.ds(gbase + NQ * NSAMPLE, flen)])
    pltpu.sync_copy(gz_v, gxyz_hbm.at[pl.ds(gbase + 2 * NQ * NSAMPLE, flen)])


def _gather_body(feat_hbm, idx_hbm, out_hbm, idx_v,
                 row0_v, row1_v, out0_v, out1_v, rsem, osem):
    wid = _wid()
    b = wid // TILES_PER_BATCH
    cbase = (wid % TILES_PER_BATCH) * C_PER_TILE
    rows = [row0_v, row1_v]
    outs = [out0_v, out1_v]

    def _row_copy(c, buf):
        src = feat_hbm.at[pl.ds((b * C + cbase + c) * N, N)]
        return pltpu.async_copy(src, buf, rsem)

    def _out_copy(c, qc, buf):
        obase = ((b * C + cbase + c) * NQ + qc * QCHUNK) * NSAMPLE
        return pltpu.async_copy(buf, out_hbm.at[pl.ds(obase, CLEN)], osem)

    for qc in range(NQ // QCHUNK):
        pltpu.sync_copy(
            idx_hbm.at[pl.ds(b * NQ * NSAMPLE + qc * CLEN, CLEN)], idx_v)
        rd = {0: _row_copy(0, rows[0])}
        od = {}
        for c in range(C_PER_TILE):
            rd[c].wait()
            if c + 1 < C_PER_TILE:
                rd[c + 1] = _row_copy(c + 1, rows[(c + 1) % 2])
            if c - 2 in od:
                od[c - 2].wait()
            row_buf = rows[c % 2]
            out_buf = outs[c % 2]

            @plsc.parallel_loop(0, CLEN, L, unroll=16)
            def gloop(o):
                idxv = idx_v[pl.ds(o, L)]
                out_buf[pl.ds(o, L)] = plsc.load_gather(row_buf, [idxv])
            od[c] = _out_copy(c, qc, out_buf)
        od[C_PER_TILE - 2].wait()
        od[C_PER_TILE - 1].wait()


@jax.jit
def _run(coords, q_flat, key_features):
    select = pl.kernel(
        _select_body,
        out_type=[
            jax.ShapeDtypeStruct((B * NQ * NSAMPLE,), jnp.int32),
            jax.ShapeDtypeStruct((B * NQ * NSAMPLE,), jnp.int32),
            jax.ShapeDtypeStruct((B * 3 * NQ * NSAMPLE,), jnp.float32),
        ],
        mesh=_mesh(),
        compiler_params=_params(),
        scratch_types=[
            pltpu.VMEM((N * 3,), jnp.float32),
            pltpu.VMEM((N,), jnp.float32),
            pltpu.VMEM((N,), jnp.float32),
            pltpu.VMEM((N,), jnp.float32),
            pltpu.VMEM((Q_PER_TILE * 6,), jnp.float32),
            pltpu.VMEM((Q_PER_TILE * NSAMPLE + NSAMPLE,), jnp.int32),
            pltpu.VMEM((Q_PER_TILE * NSAMPLE,), jnp.int32),
            pltpu.VMEM((Q_PER_TILE * NSAMPLE,), jnp.float32),
            pltpu.VMEM((Q_PER_TILE * NSAMPLE,), jnp.float32),
            pltpu.VMEM((Q_PER_TILE * NSAMPLE,), jnp.float32),
        ],
    )
    idx, mask_i, gxyz = select(coords, q_flat)

    gather = pl.kernel(
        _gather_body,
        out_type=jax.ShapeDtypeStruct((B * C * NQ * NSAMPLE,), jnp.float32),
        mesh=_mesh(),
        compiler_params=_params(),
        scratch_types=[
            pltpu.VMEM((CLEN,), jnp.int32),
            pltpu.VMEM((N,), jnp.float32),
            pltpu.VMEM((N,), jnp.float32),
            pltpu.VMEM((CLEN,), jnp.float32),
            pltpu.VMEM((CLEN,), jnp.float32),
            pltpu.SemaphoreType.DMA,
            pltpu.SemaphoreType.DMA,
        ],
    )
    gfeat = gather(key_features, idx)

    gxyz = gxyz.reshape(B, 3, NQ, NSAMPLE)
    gfeat = gfeat.reshape(B, C, NQ, NSAMPLE)
    mask = mask_i.reshape(B, NQ, NSAMPLE).astype(bool)
    return gxyz, gfeat, mask


def kernel(key_xyz, key_features, query_xyz):
    return _run(key_xyz.reshape(-1), query_xyz.reshape(-1),
                key_features.reshape(-1))


# speedup vs baseline: 28.7674x; 1.1386x over previous
import jax
import jax.numpy as jnp
from jax import lax
from jax.experimental import pallas as pl
from jax.experimental.pallas import tpu as pltpu
from jax.experimental.pallas import tpu_sc as plsc

NSAMPLE = 64
L = 16
NUM_TILES = 32
B, N, NQ, C = 4, 8192, 1024, 128
Q_PER_TILE = NQ * B // NUM_TILES
TILES_PER_BATCH = NUM_TILES // B
C_PER_TILE = C // TILES_PER_BATCH
NKV = N // L
U = 4
QCHUNK = 256
CLEN = QCHUNK * NSAMPLE


def _mesh():
    return plsc.VectorSubcoreMesh(core_axis_name="c", subcore_axis_name="s")


def _params():
    return pltpu.CompilerParams(needs_layout_passes=False)


def _wid():
    return lax.axis_index("s") * 2 + lax.axis_index("c")


def _select_body(coords_hbm, query_hbm, idx_hbm, mask_hbm, gxyz_hbm,
                 cint_v, xs_v, ys_v, zs_v, q_v, idx_v, mask_v,
                 gx_v, gy_v, gz_v):
    wid = _wid()
    b = wid // TILES_PER_BATCH
    qbase = (wid % TILES_PER_BATCH) * Q_PER_TILE

    pltpu.sync_copy(coords_hbm.at[pl.ds(b * N * 3, N * 3)], cint_v)
    pltpu.sync_copy(
        query_hbm.at[pl.ds((b * NQ + qbase) * 6, Q_PER_TILE * 6)], q_v)

    lane = jnp.arange(L, dtype=jnp.int32)
    zeros_i = jnp.zeros((L,), jnp.int32)

    @plsc.parallel_loop(0, N, L, unroll=8)
    def dloop(kb):
        idx3 = (kb + lane) * 3
        xs_v[pl.ds(kb, L)] = plsc.load_gather(cint_v, [idx3])
        ys_v[pl.ds(kb, L)] = plsc.load_gather(cint_v, [idx3 + 1])
        zs_v[pl.ds(kb, L)] = plsc.load_gather(cint_v, [idx3 + 2])

    def qloop(q, _):
        qsplat = jnp.full((L,), q, jnp.int32)
        q6 = qsplat * 6

        def qval(d):
            return plsc.load_gather(q_v, [q6 + d])

        cx, cy, cz = qval(0), qval(1), qval(2)
        hx, hy, hz = 0.5 * qval(3), 0.5 * qval(4), 0.5 * qval(5)
        obase = q * NSAMPLE

        for j in range(NSAMPLE // L):
            idx_v[pl.ds(obase + j * L, L)] = zeros_i

        def cond(carry):
            i, cnt = carry
            return jnp.logical_and(i < NKV, cnt < NSAMPLE)

        def body(carry):
            i, cnt = carry
            kb = i * L
            insides = []
            pcs = []
            for u in range(U):
                xv = xs_v[pl.ds(kb + u * L, L)]
                yv = ys_v[pl.ds(kb + u * L, L)]
                zv = zs_v[pl.ds(kb + u * L, L)]
                inside = jnp.logical_and(
                    jnp.logical_and(jnp.abs(xv - cx) <= hx,
                                    jnp.abs(yv - cy) <= hy),
                    jnp.abs(zv - cz) <= hz)
                insides.append(inside)
                pcs.append(plsc.all_reduce_population_count(inside))
            tot_v = pcs[0] + pcs[1] + pcs[2] + pcs[3]
            tot = tot_v[0]

            @pl.when(tot > 0)
            def _():
                off = obase + cnt
                for u in range(U):
                    plsc.store_compressed(idx_v.at[pl.ds(off, L)],
                                          kb + u * L + lane, mask=insides[u])
                    if u + 1 < U:
                        off = off + pcs[u][0]

            return i + U, cnt + tot

        _, cnt = lax.while_loop(cond, body, (jnp.int32(0), jnp.int32(0)))
        cntv = jnp.full((L,), jnp.minimum(cnt, NSAMPLE), jnp.int32)

        for j in range(NSAMPLE // L):
            s_ids = j * L + lane
            idxv = idx_v[pl.ds(obase + j * L, L)]
            invalid = s_ids >= cntv
            if j == 0:
                invalid = jnp.logical_and(invalid, s_ids != 0)
            mask_v[q, pl.ds(j * L, L)] = invalid.astype(jnp.int32)
            gx_v[q, pl.ds(j * L, L)] = plsc.load_gather(xs_v, [idxv]) - cx
            gy_v[q, pl.ds(j * L, L)] = plsc.load_gather(ys_v, [idxv]) - cy
            gz_v[q, pl.ds(j * L, L)] = plsc.load_gather(zs_v, [idxv]) - cz
        return 0

    lax.fori_loop(0, Q_PER_TILE, qloop, 0)

    flen = Q_PER_TILE * NSAMPLE
    fbase = (b * NQ + qbase) * NSAMPLE
    pltpu.sync_copy(idx_v.at[pl.ds(0, flen)], idx_hbm.at[pl.ds(fbase, flen)])
    qsl = pl.ds(qbase, Q_PER_TILE)
    pltpu.sync_copy(mask_v, mask_hbm.at[b, qsl, :])
    pltpu.sync_copy(gx_v, gxyz_hbm.at[b, 0, qsl, :])
    pltpu.sync_copy(gy_v, gxyz_hbm.at[b, 1, qsl, :])
    pltpu.sync_copy(gz_v, gxyz_hbm.at[b, 2, qsl, :])


def _gather_body(feat_hbm, idx_hbm, out_hbm, idx_v,
                 row0_v, row1_v, out0_v, out1_v, rsem, osem):
    wid = _wid()
    b = wid // TILES_PER_BATCH
    cbase = (wid % TILES_PER_BATCH) * C_PER_TILE
    rows = [row0_v, row1_v]
    outs = [out0_v, out1_v]

    def _row_copy(c, buf):
        src = feat_hbm.at[pl.ds((b * C + cbase + c) * N, N)]
        return pltpu.async_copy(src, buf, rsem)

    def _out_copy(c, qc, buf):
        dst = out_hbm.at[b, cbase + c, pl.ds(qc * QCHUNK, QCHUNK), :]
        return pltpu.async_copy(buf, dst, osem)

    def qc_body(qc, _):
        pltpu.sync_copy(
            idx_hbm.at[pl.ds(b * NQ * NSAMPLE + qc * CLEN, CLEN)], idx_v)
        rd = {0: _row_copy(0, rows[0])}
        od = {}
        for c in range(C_PER_TILE):
            rd[c].wait()
            if c + 1 < C_PER_TILE:
                rd[c + 1] = _row_copy(c + 1, rows[(c + 1) % 2])
            if c - 2 in od:
                od[c - 2].wait()
            row_buf = rows[c % 2]
            out_buf = outs[c % 2]

            @plsc.parallel_loop(0, QCHUNK, 1, unroll=4)
            def gloop(r):
                for j in range(NSAMPLE // L):
                    idxv = idx_v[pl.ds(r * NSAMPLE + j * L, L)]
                    out_buf[r, pl.ds(j * L, L)] = plsc.load_gather(
                        row_buf, [idxv])
            od[c] = _out_copy(c, qc, out_buf)
        od[C_PER_TILE - 2].wait()
        od[C_PER_TILE - 1].wait()
        return 0

    lax.fori_loop(0, NQ // QCHUNK, qc_body, 0)


@jax.jit
def _run(coords, q_flat, key_features):
    select = pl.kernel(
        _select_body,
        out_type=[
            jax.ShapeDtypeStruct((B * NQ * NSAMPLE,), jnp.int32),
            jax.ShapeDtypeStruct((B, NQ, NSAMPLE), jnp.int32),
            jax.ShapeDtypeStruct((B, 3, NQ, NSAMPLE), jnp.float32),
        ],
        mesh=_mesh(),
        compiler_params=_params(),
        scratch_types=[
            pltpu.VMEM((N * 3,), jnp.float32),
            pltpu.VMEM((N,), jnp.float32),
            pltpu.VMEM((N,), jnp.float32),
            pltpu.VMEM((N,), jnp.float32),
            pltpu.VMEM((Q_PER_TILE * 6,), jnp.float32),
            pltpu.VMEM((Q_PER_TILE * NSAMPLE + NSAMPLE,), jnp.int32),
            pltpu.VMEM((Q_PER_TILE, NSAMPLE), jnp.int32),
            pltpu.VMEM((Q_PER_TILE, NSAMPLE), jnp.float32),
            pltpu.VMEM((Q_PER_TILE, NSAMPLE), jnp.float32),
            pltpu.VMEM((Q_PER_TILE, NSAMPLE), jnp.float32),
        ],
    )
    idx, mask_i, gxyz = select(coords, q_flat)

    gather = pl.kernel(
        _gather_body,
        out_type=jax.ShapeDtypeStruct((B, C, NQ, NSAMPLE), jnp.float32),
        mesh=_mesh(),
        compiler_params=_params(),
        scratch_types=[
            pltpu.VMEM((CLEN,), jnp.int32),
            pltpu.VMEM((N,), jnp.float32),
            pltpu.VMEM((N,), jnp.float32),
            pltpu.VMEM((QCHUNK, NSAMPLE), jnp.float32),
            pltpu.VMEM((QCHUNK, NSAMPLE), jnp.float32),
            pltpu.SemaphoreType.DMA,
            pltpu.SemaphoreType.DMA,
        ],
    )
    gfeat = gather(key_features, idx)

    return gxyz, gfeat, mask_i.astype(bool)


def kernel(key_xyz, key_features, query_xyz):
    return _run(key_xyz.reshape(-1), query_xyz.reshape(-1),
                key_features.reshape(-1))


# speedup vs baseline: 37.0642x; 1.2884x over previous
import jax
import jax.numpy as jnp
from jax import lax
from jax.experimental import pallas as pl
from jax.experimental.pallas import tpu as pltpu
from jax.experimental.pallas import tpu_sc as plsc

NSAMPLE = 64
L = 16
NUM_TILES = 32
B, N, NQ, C = 4, 8192, 1024, 128
Q_PER_TILE = NQ * B // NUM_TILES
TILES_PER_BATCH = NUM_TILES // B
C_PER_TILE = C // TILES_PER_BATCH
NKV = N // L
U = 4
QCHUNK = 256
CLEN = QCHUNK * NSAMPLE


def _mesh():
    return plsc.VectorSubcoreMesh(core_axis_name="c", subcore_axis_name="s")


def _params():
    return pltpu.CompilerParams(needs_layout_passes=False)


def _wid():
    return lax.axis_index("s") * 2 + lax.axis_index("c")


def _select_body(coords_hbm, query_hbm, idx_hbm, mask_hbm, gxyz_hbm,
                 cint_v, xs_v, ys_v, zs_v, q_v, idx_v, mask_v,
                 gx_v, gy_v, gz_v, t_v, ti_v):
    wid = _wid()
    b = wid // TILES_PER_BATCH
    qbase = (wid % TILES_PER_BATCH) * Q_PER_TILE

    pltpu.sync_copy(coords_hbm.at[pl.ds(b * N * 3, N * 3)], cint_v)
    pltpu.sync_copy(
        query_hbm.at[pl.ds((b * NQ + qbase) * 6, Q_PER_TILE * 6)], q_v)

    lane = jnp.arange(L, dtype=jnp.int32)
    zeros_i = jnp.zeros((L,), jnp.int32)

    @plsc.parallel_loop(0, N, L, unroll=8)
    def dloop(kb):
        idx3 = (kb + lane) * 3
        xs_v[pl.ds(kb, L)] = plsc.load_gather(cint_v, [idx3])
        ys_v[pl.ds(kb, L)] = plsc.load_gather(cint_v, [idx3 + 1])
        zs_v[pl.ds(kb, L)] = plsc.load_gather(cint_v, [idx3 + 2])

    def qloop(q, _):
        qsplat = jnp.full((L,), q, jnp.int32)
        q6 = qsplat * 6

        def qval(d):
            return plsc.load_gather(q_v, [q6 + d])

        cx, cy, cz = qval(0), qval(1), qval(2)
        hx, hy, hz = 0.5 * qval(3), 0.5 * qval(4), 0.5 * qval(5)
        obase = q * NSAMPLE

        for j in range(NSAMPLE // L):
            idx_v[pl.ds(obase + j * L, L)] = zeros_i

        def cond(carry):
            i, cnt = carry
            return jnp.logical_and(i < NKV, cnt < NSAMPLE)

        def body(carry):
            i, cnt = carry
            kb = i * L
            insides = []
            pcs = []
            for u in range(U):
                xv = xs_v[pl.ds(kb + u * L, L)]
                yv = ys_v[pl.ds(kb + u * L, L)]
                zv = zs_v[pl.ds(kb + u * L, L)]
                inside = jnp.logical_and(
                    jnp.logical_and(jnp.abs(xv - cx) <= hx,
                                    jnp.abs(yv - cy) <= hy),
                    jnp.abs(zv - cz) <= hz)
                insides.append(inside)
                pcs.append(plsc.all_reduce_population_count(inside))
            tot_v = pcs[0] + pcs[1] + pcs[2] + pcs[3]
            tot = tot_v[0]

            @pl.when(tot > 0)
            def _():
                off = obase + cnt
                for u in range(U):
                    plsc.store_compressed(idx_v.at[pl.ds(off, L)],
                                          kb + u * L + lane, mask=insides[u])
                    if u + 1 < U:
                        off = off + pcs[u][0]

            return i + U, cnt + tot

        _, cnt = lax.while_loop(cond, body, (jnp.int32(0), jnp.int32(0)))
        cntv = jnp.full((L,), jnp.minimum(cnt, NSAMPLE), jnp.int32)

        for j in range(NSAMPLE // L):
            s_ids = j * L + lane
            idxv = idx_v[pl.ds(obase + j * L, L)]
            invalid = s_ids >= cntv
            if j == 0:
                invalid = jnp.logical_and(invalid, s_ids != 0)
            mask_v[pl.ds(obase + j * L, L)] = invalid.astype(jnp.int32)
            gx_v[pl.ds(obase + j * L, L)] = plsc.load_gather(xs_v, [idxv]) - cx
            gy_v[pl.ds(obase + j * L, L)] = plsc.load_gather(ys_v, [idxv]) - cy
            gz_v[pl.ds(obase + j * L, L)] = plsc.load_gather(zs_v, [idxv]) - cz
        return 0

    lax.fori_loop(0, Q_PER_TILE, qloop, 0)

    flen = Q_PER_TILE * NSAMPLE
    fbase = (b * NQ + qbase) * NSAMPLE
    pltpu.sync_copy(idx_v.at[pl.ds(0, flen)], idx_hbm.at[pl.ds(fbase, flen)])

    qsl = pl.ds(qbase, Q_PER_TILE)

    def _emit(src_v, dst):
        @plsc.parallel_loop(0, NSAMPLE, 1, unroll=2)
        def tloop(s):
            for qb in range(Q_PER_TILE // L):
                gi = (qb * L + lane) * NSAMPLE + s
                t_v[s, pl.ds(qb * L, L)] = plsc.load_gather(src_v, [gi])
        pltpu.sync_copy(t_v, dst)

    @plsc.parallel_loop(0, NSAMPLE, 1, unroll=2)
    def mloop(s):
        for qb in range(Q_PER_TILE // L):
            gi = (qb * L + lane) * NSAMPLE + s
            ti_v[s, pl.ds(qb * L, L)] = plsc.load_gather(mask_v, [gi])
    pltpu.sync_copy(ti_v, mask_hbm.at[b, :, qsl])
    _emit(gx_v, gxyz_hbm.at[b, 0, :, qsl])
    _emit(gy_v, gxyz_hbm.at[b, 1, :, qsl])
    _emit(gz_v, gxyz_hbm.at[b, 2, :, qsl])


def _gather_body(feat_hbm, idx_hbm, out_hbm, idx_v, idxt_v,
                 row0_v, row1_v, out0_v, out1_v, rsem, osem):
    wid = _wid()
    b = wid // TILES_PER_BATCH
    cbase = (wid % TILES_PER_BATCH) * C_PER_TILE
    rows = [row0_v, row1_v]
    outs = [out0_v, out1_v]

    def _row_copy(c, buf):
        src = feat_hbm.at[pl.ds((b * C + cbase + c) * N, N)]
        return pltpu.async_copy(src, buf, rsem)

    def _out_copy(c, qc, buf):
        dst = out_hbm.at[b, cbase + c, :, pl.ds(qc * QCHUNK, QCHUNK)]
        return pltpu.async_copy(buf, dst, osem)

    lane = jnp.arange(L, dtype=jnp.int32)

    def qc_body(qc, _):
        pltpu.sync_copy(
            idx_hbm.at[pl.ds(b * NQ * NSAMPLE + qc * CLEN, CLEN)], idx_v)

        @plsc.parallel_loop(0, NSAMPLE, 1, unroll=2)
        def itloop(s):
            for qb in range(QCHUNK // L):
                gi = (qb * L + lane) * NSAMPLE + s
                idxt_v[s, pl.ds(qb * L, L)] = plsc.load_gather(idx_v, [gi])

        rd = {0: _row_copy(0, rows[0])}
        od = {}
        for c in range(C_PER_TILE):
            rd[c].wait()
            if c + 1 < C_PER_TILE:
                rd[c + 1] = _row_copy(c + 1, rows[(c + 1) % 2])
            if c - 2 in od:
                od[c - 2].wait()
            row_buf = rows[c % 2]
            out_buf = outs[c % 2]

            @plsc.parallel_loop(0, NSAMPLE, 1, unroll=2)
            def gloop(s):
                for qb in range(QCHUNK // L):
                    idxv = idxt_v[s, pl.ds(qb * L, L)]
                    out_buf[s, pl.ds(qb * L, L)] = plsc.load_gather(
                        row_buf, [idxv])
            od[c] = _out_copy(c, qc, out_buf)
        od[C_PER_TILE - 2].wait()
        od[C_PER_TILE - 1].wait()
        return 0

    lax.fori_loop(0, NQ // QCHUNK, qc_body, 0)


@jax.jit
def _run(coords, q_flat, key_features):
    select = pl.kernel(
        _select_body,
        out_type=[
            jax.ShapeDtypeStruct((B * NQ * NSAMPLE,), jnp.int32),
            jax.ShapeDtypeStruct((B, NSAMPLE, NQ), jnp.int32),
            jax.ShapeDtypeStruct((B, 3, NSAMPLE, NQ), jnp.float32),
        ],
        mesh=_mesh(),
        compiler_params=_params(),
        scratch_types=[
            pltpu.VMEM((N * 3,), jnp.float32),
            pltpu.VMEM((N,), jnp.float32),
            pltpu.VMEM((N,), jnp.float32),
            pltpu.VMEM((N,), jnp.float32),
            pltpu.VMEM((Q_PER_TILE * 6,), jnp.float32),
            pltpu.VMEM((Q_PER_TILE * NSAMPLE + NSAMPLE,), jnp.int32),
            pltpu.VMEM((Q_PER_TILE * NSAMPLE,), jnp.int32),
            pltpu.VMEM((Q_PER_TILE * NSAMPLE,), jnp.float32),
            pltpu.VMEM((Q_PER_TILE * NSAMPLE,), jnp.float32),
            pltpu.VMEM((Q_PER_TILE * NSAMPLE,), jnp.float32),
            pltpu.VMEM((NSAMPLE, Q_PER_TILE), jnp.float32),
            pltpu.VMEM((NSAMPLE, Q_PER_TILE), jnp.int32),
        ],
    )
    idx, mask_i, gxyz = select(coords, q_flat)

    gather = pl.kernel(
        _gather_body,
        out_type=jax.ShapeDtypeStruct((B, C, NSAMPLE, NQ), jnp.float32),
        mesh=_mesh(),
        compiler_params=_params(),
        scratch_types=[
            pltpu.VMEM((CLEN,), jnp.int32),
            pltpu.VMEM((NSAMPLE, QCHUNK), jnp.int32),
            pltpu.VMEM((N,), jnp.float32),
            pltpu.VMEM((N,), jnp.float32),
            pltpu.VMEM((NSAMPLE, QCHUNK), jnp.float32),
            pltpu.VMEM((NSAMPLE, QCHUNK), jnp.float32),
            pltpu.SemaphoreType.DMA,
            pltpu.SemaphoreType.DMA,
        ],
    )
    gfeat = gather(key_features, idx)

    gxyz = jnp.transpose(gxyz, (0, 1, 3, 2))
    gfeat = jnp.transpose(gfeat, (0, 1, 3, 2))
    mask = jnp.transpose(mask_i, (0, 2, 1)).astype(bool)
    return gxyz, gfeat, mask


def kernel(key_xyz, key_features, query_xyz):
    return _run(key_xyz.reshape(-1), query_xyz.reshape(-1),
                key_features.reshape(-1))


# speedup vs baseline: 43.2883x; 1.1679x over previous
import jax
import jax.numpy as jnp
from jax import lax
from jax.experimental import pallas as pl
from jax.experimental.pallas import tpu as pltpu
from jax.experimental.pallas import tpu_sc as plsc

NSAMPLE = 64
L = 16
NUM_TILES = 32
B, N, NQ, C = 4, 8192, 1024, 128
Q_PER_TILE = NQ * B // NUM_TILES
TILES_PER_BATCH = NUM_TILES // B
C_PER_TILE = C // TILES_PER_BATCH
NKV = N // L
U = 8
QCHUNK = 256
CLEN = QCHUNK * NSAMPLE


def _mesh():
    return plsc.VectorSubcoreMesh(core_axis_name="c", subcore_axis_name="s")


def _params():
    return pltpu.CompilerParams(needs_layout_passes=False)


def _wid():
    return lax.axis_index("s") * 2 + lax.axis_index("c")


def _select_body(coords_hbm, query_hbm, idx_hbm, mask_hbm, gxyz_hbm,
                 cint_v, xs_v, ys_v, zs_v, q_v, idx_v, mask_v,
                 gx_v, gy_v, gz_v, t_v, ti_v):
    wid = _wid()
    b = wid // TILES_PER_BATCH
    qbase = (wid % TILES_PER_BATCH) * Q_PER_TILE

    pltpu.sync_copy(coords_hbm.at[pl.ds(b * N * 3, N * 3)], cint_v)
    pltpu.sync_copy(
        query_hbm.at[pl.ds((b * NQ + qbase) * 6, Q_PER_TILE * 6)], q_v)

    lane = jnp.arange(L, dtype=jnp.int32)
    zeros_i = jnp.zeros((L,), jnp.int32)

    @plsc.parallel_loop(0, N, L, unroll=8)
    def dloop(kb):
        idx3 = (kb + lane) * 3
        xs_v[pl.ds(kb, L)] = plsc.load_gather(cint_v, [idx3])
        ys_v[pl.ds(kb, L)] = plsc.load_gather(cint_v, [idx3 + 1])
        zs_v[pl.ds(kb, L)] = plsc.load_gather(cint_v, [idx3 + 2])

    def qloop(q, _):
        qsplat = jnp.full((L,), q, jnp.int32)
        q6 = qsplat * 6

        def qval(d):
            return plsc.load_gather(q_v, [q6 + d])

        cx, cy, cz = qval(0), qval(1), qval(2)
        hx, hy, hz = 0.5 * qval(3), 0.5 * qval(4), 0.5 * qval(5)
        obase = q * NSAMPLE

        for j in range(NSAMPLE // L):
            idx_v[pl.ds(obase + j * L, L)] = zeros_i

        def cond(carry):
            i, cnt = carry
            return jnp.logical_and(i < NKV, cnt < NSAMPLE)

        def body(carry):
            i, cnt = carry
            kb = i * L
            insides = []
            pcs = []
            for u in range(U):
                xv = xs_v[pl.ds(kb + u * L, L)]
                yv = ys_v[pl.ds(kb + u * L, L)]
                zv = zs_v[pl.ds(kb + u * L, L)]
                inside = jnp.logical_and(
                    jnp.logical_and(jnp.abs(xv - cx) <= hx,
                                    jnp.abs(yv - cy) <= hy),
                    jnp.abs(zv - cz) <= hz)
                insides.append(inside)
                pcs.append(plsc.all_reduce_population_count(inside))
            tot_v = ((pcs[0] + pcs[1]) + (pcs[2] + pcs[3])
                     + ((pcs[4] + pcs[5]) + (pcs[6] + pcs[7])))
            tot = tot_v[0]

            @pl.when(tot > 0)
            def _():
                off = obase + cnt
                for u in range(U):
                    plsc.store_compressed(idx_v.at[pl.ds(off, L)],
                                          kb + u * L + lane, mask=insides[u])
                    if u + 1 < U:
                        off = off + pcs[u][0]

            return i + U, cnt + tot

        _, cnt = lax.while_loop(cond, body, (jnp.int32(0), jnp.int32(0)))
        cntv = jnp.full((L,), jnp.minimum(cnt, NSAMPLE), jnp.int32)

        for j in range(NSAMPLE // L):
            s_ids = j * L + lane
            idxv = idx_v[pl.ds(obase + j * L, L)]
            invalid = s_ids >= cntv
            if j == 0:
                invalid = jnp.logical_and(invalid, s_ids != 0)
            mask_v[pl.ds(obase + j * L, L)] = invalid.astype(jnp.int32)
            gx_v[pl.ds(obase + j * L, L)] = plsc.load_gather(xs_v, [idxv]) - cx
            gy_v[pl.ds(obase + j * L, L)] = plsc.load_gather(ys_v, [idxv]) - cy
            gz_v[pl.ds(obase + j * L, L)] = plsc.load_gather(zs_v, [idxv]) - cz
        return 0

    lax.fori_loop(0, Q_PER_TILE, qloop, 0)

    flen = Q_PER_TILE * NSAMPLE
    fbase = (b * NQ + qbase) * NSAMPLE
    pltpu.sync_copy(idx_v.at[pl.ds(0, flen)], idx_hbm.at[pl.ds(fbase, flen)])

    qsl = pl.ds(qbase, Q_PER_TILE)

    def _emit(src_v, dst):
        @plsc.parallel_loop(0, NSAMPLE, 1, unroll=2)
        def tloop(s):
            for qb in range(Q_PER_TILE // L):
                gi = (qb * L + lane) * NSAMPLE + s
                t_v[s, pl.ds(qb * L, L)] = plsc.load_gather(src_v, [gi])
        pltpu.sync_copy(t_v, dst)

    @plsc.parallel_loop(0, NSAMPLE, 1, unroll=2)
    def mloop(s):
        for qb in range(Q_PER_TILE // L):
            gi = (qb * L + lane) * NSAMPLE + s
            ti_v[s, pl.ds(qb * L, L)] = plsc.load_gather(mask_v, [gi])
    pltpu.sync_copy(ti_v, mask_hbm.at[b, :, qsl])
    _emit(gx_v, gxyz_hbm.at[b, 0, :, qsl])
    _emit(gy_v, gxyz_hbm.at[b, 1, :, qsl])
    _emit(gz_v, gxyz_hbm.at[b, 2, :, qsl])


def _gather_body(feat_hbm, idx_hbm, out_hbm, idx_v, idxt_v,
                 row0_v, row1_v, out0_v, out1_v, rsem, osem):
    wid = _wid()
    b = wid // TILES_PER_BATCH
    cbase = (wid % TILES_PER_BATCH) * C_PER_TILE
    rows = [row0_v, row1_v]
    outs = [out0_v, out1_v]

    def _row_copy(c, buf):
        src = feat_hbm.at[pl.ds((b * C + cbase + c) * N, N)]
        return pltpu.async_copy(src, buf, rsem)

    def _out_copy(c, qc, buf):
        dst = out_hbm.at[b, cbase + c, :, pl.ds(qc * QCHUNK, QCHUNK)]
        return pltpu.async_copy(buf, dst, osem)

    lane = jnp.arange(L, dtype=jnp.int32)

    def qc_body(qc, _):
        pltpu.sync_copy(
            idx_hbm.at[pl.ds(b * NQ * NSAMPLE + qc * CLEN, CLEN)], idx_v)

        @plsc.parallel_loop(0, NSAMPLE, 1, unroll=2)
        def itloop(s):
            for qb in range(QCHUNK // L):
                gi = (qb * L + lane) * NSAMPLE + s
                idxt_v[s, pl.ds(qb * L, L)] = plsc.load_gather(idx_v, [gi])

        rd = {0: _row_copy(0, rows[0])}
        od = {}
        for c in range(C_PER_TILE):
            rd[c].wait()
            if c + 1 < C_PER_TILE:
                rd[c + 1] = _row_copy(c + 1, rows[(c + 1) % 2])
            if c - 2 in od:
                od[c - 2].wait()
            row_buf = rows[c % 2]
            out_buf = outs[c % 2]

            @plsc.parallel_loop(0, NSAMPLE, 1, unroll=2)
            def gloop(s):
                for qb in range(QCHUNK // L):
                    idxv = idxt_v[s, pl.ds(qb * L, L)]
                    out_buf[s, pl.ds(qb * L, L)] = plsc.load_gather(
                        row_buf, [idxv])
            od[c] = _out_copy(c, qc, out_buf)
        od[C_PER_TILE - 2].wait()
        od[C_PER_TILE - 1].wait()
        return 0

    lax.fori_loop(0, NQ // QCHUNK, qc_body, 0)


@jax.jit
def _run(coords, q_flat, key_features):
    select = pl.kernel(
        _select_body,
        out_type=[
            jax.ShapeDtypeStruct((B * NQ * NSAMPLE,), jnp.int32),
            jax.ShapeDtypeStruct((B, NSAMPLE, NQ), jnp.int32),
            jax.ShapeDtypeStruct((B, 3, NSAMPLE, NQ), jnp.float32),
        ],
        mesh=_mesh(),
        compiler_params=_params(),
        scratch_types=[
            pltpu.VMEM((N * 3,), jnp.float32),
            pltpu.VMEM((N,), jnp.float32),
            pltpu.VMEM((N,), jnp.float32),
            pltpu.VMEM((N,), jnp.float32),
            pltpu.VMEM((Q_PER_TILE * 6,), jnp.float32),
            pltpu.VMEM((Q_PER_TILE * NSAMPLE + NSAMPLE,), jnp.int32),
            pltpu.VMEM((Q_PER_TILE * NSAMPLE,), jnp.int32),
            pltpu.VMEM((Q_PER_TILE * NSAMPLE,), jnp.float32),
            pltpu.VMEM((Q_PER_TILE * NSAMPLE,), jnp.float32),
            pltpu.VMEM((Q_PER_TILE * NSAMPLE,), jnp.float32),
            pltpu.VMEM((NSAMPLE, Q_PER_TILE), jnp.float32),
            pltpu.VMEM((NSAMPLE, Q_PER_TILE), jnp.int32),
        ],
    )
    idx, mask_i, gxyz = select(coords, q_flat)

    gather = pl.kernel(
        _gather_body,
        out_type=jax.ShapeDtypeStruct((B, C, NSAMPLE, NQ), jnp.float32),
        mesh=_mesh(),
        compiler_params=_params(),
        scratch_types=[
            pltpu.VMEM((CLEN,), jnp.int32),
            pltpu.VMEM((NSAMPLE, QCHUNK), jnp.int32),
            pltpu.VMEM((N,), jnp.float32),
            pltpu.VMEM((N,), jnp.float32),
            pltpu.VMEM((NSAMPLE, QCHUNK), jnp.float32),
            pltpu.VMEM((NSAMPLE, QCHUNK), jnp.float32),
            pltpu.SemaphoreType.DMA,
            pltpu.SemaphoreType.DMA,
        ],
    )
    gfeat = gather(key_features, idx)

    gxyz = jnp.transpose(gxyz, (0, 1, 3, 2))
    gfeat = jnp.transpose(gfeat, (0, 1, 3, 2))
    mask = jnp.transpose(mask_i, (0, 2, 1)).astype(bool)
    return gxyz, gfeat, mask


def kernel(key_xyz, key_features, query_xyz):
    return _run(key_xyz.reshape(-1), query_xyz.reshape(-1),
                key_features.reshape(-1))


# speedup vs baseline: 46.3795x; 1.0714x over previous
import jax
import jax.numpy as jnp
from jax import lax
from jax.experimental import pallas as pl
from jax.experimental.pallas import tpu as pltpu
from jax.experimental.pallas import tpu_sc as plsc

NSAMPLE = 64
L = 16
NUM_TILES = 32
B, N, NQ, C = 4, 8192, 1024, 128
Q_PER_TILE = NQ * B // NUM_TILES
TILES_PER_BATCH = NUM_TILES // B
C_PER_TILE = C // TILES_PER_BATCH
NKV = N // L
U = 8
QCHUNK = 512
CLEN = QCHUNK * NSAMPLE


def _mesh():
    return plsc.VectorSubcoreMesh(core_axis_name="c", subcore_axis_name="s")


def _params():
    return pltpu.CompilerParams(needs_layout_passes=False)


def _wid():
    return lax.axis_index("s") * 2 + lax.axis_index("c")


def _select_body(coords_hbm, query_hbm, idx_hbm, mask_hbm, gxyz_hbm,
                 cint_v, xs_v, ys_v, zs_v, q_v, idx_v, mask_v,
                 gx_v, gy_v, gz_v, t_v, ti_v):
    wid = _wid()
    b = wid // TILES_PER_BATCH
    qbase = (wid % TILES_PER_BATCH) * Q_PER_TILE

    pltpu.sync_copy(coords_hbm.at[pl.ds(b * N * 3, N * 3)], cint_v)
    pltpu.sync_copy(
        query_hbm.at[pl.ds((b * NQ + qbase) * 6, Q_PER_TILE * 6)], q_v)

    lane = jnp.arange(L, dtype=jnp.int32)
    zeros_i = jnp.zeros((L,), jnp.int32)

    @plsc.parallel_loop(0, N, L, unroll=8)
    def dloop(kb):
        idx3 = (kb + lane) * 3
        xs_v[pl.ds(kb, L)] = plsc.load_gather(cint_v, [idx3])
        ys_v[pl.ds(kb, L)] = plsc.load_gather(cint_v, [idx3 + 1])
        zs_v[pl.ds(kb, L)] = plsc.load_gather(cint_v, [idx3 + 2])

    def qloop(q, _):
        qsplat = jnp.full((L,), q, jnp.int32)
        q6 = qsplat * 6

        def qval(d):
            return plsc.load_gather(q_v, [q6 + d])

        cx, cy, cz = qval(0), qval(1), qval(2)
        hx, hy, hz = 0.5 * qval(3), 0.5 * qval(4), 0.5 * qval(5)
        obase = q * NSAMPLE

        for j in range(NSAMPLE // L):
            idx_v[pl.ds(obase + j * L, L)] = zeros_i

        def cond(carry):
            i, cnt = carry
            return jnp.logical_and(i < NKV, cnt < NSAMPLE)

        def body(carry):
            i, cnt = carry
            kb = i * L
            insides = []
            pcs = []
            for u in range(U):
                xv = xs_v[pl.ds(kb + u * L, L)]
                yv = ys_v[pl.ds(kb + u * L, L)]
                zv = zs_v[pl.ds(kb + u * L, L)]
                inside = jnp.logical_and(
                    jnp.logical_and(jnp.abs(xv - cx) <= hx,
                                    jnp.abs(yv - cy) <= hy),
                    jnp.abs(zv - cz) <= hz)
                insides.append(inside)
                pcs.append(plsc.all_reduce_population_count(inside))
            tot_v = ((pcs[0] + pcs[1]) + (pcs[2] + pcs[3])
                     + ((pcs[4] + pcs[5]) + (pcs[6] + pcs[7])))
            tot = tot_v[0]

            @pl.when(tot > 0)
            def _():
                off = obase + cnt
                for u in range(U):
                    plsc.store_compressed(idx_v.at[pl.ds(off, L)],
                                          kb + u * L + lane, mask=insides[u])
                    if u + 1 < U:
                        off = off + pcs[u][0]

            return i + U, cnt + tot

        _, cnt = lax.while_loop(cond, body, (jnp.int32(0), jnp.int32(0)))
        cntv = jnp.full((L,), jnp.minimum(cnt, NSAMPLE), jnp.int32)

        for j in range(NSAMPLE // L):
            s_ids = j * L + lane
            idxv = idx_v[pl.ds(obase + j * L, L)]
            invalid = s_ids >= cntv
            if j == 0:
                invalid = jnp.logical_and(invalid, s_ids != 0)
            mask_v[pl.ds(obase + j * L, L)] = invalid.astype(jnp.int32)
            gx_v[pl.ds(obase + j * L, L)] = plsc.load_gather(xs_v, [idxv]) - cx
            gy_v[pl.ds(obase + j * L, L)] = plsc.load_gather(ys_v, [idxv]) - cy
            gz_v[pl.ds(obase + j * L, L)] = plsc.load_gather(zs_v, [idxv]) - cz
        return 0

    lax.fori_loop(0, Q_PER_TILE, qloop, 0)

    qsl = pl.ds(qbase, Q_PER_TILE)

    def _emit(src_v, dst):
        @plsc.parallel_loop(0, NSAMPLE, 1, unroll=2)
        def tloop(s):
            for qb in range(Q_PER_TILE // L):
                gi = (qb * L + lane) * NSAMPLE + s
                t_v[s, pl.ds(qb * L, L)] = plsc.load_gather(src_v, [gi])
        pltpu.sync_copy(t_v, dst)

    @plsc.parallel_loop(0, NSAMPLE, 1, unroll=2)
    def mloop(s):
        for qb in range(Q_PER_TILE // L):
            gi = (qb * L + lane) * NSAMPLE + s
            ti_v[s, pl.ds(qb * L, L)] = plsc.load_gather(mask_v, [gi])
    pltpu.sync_copy(ti_v, mask_hbm.at[b, :, qsl])

    @plsc.parallel_loop(0, NSAMPLE, 1, unroll=2)
    def iloop(s):
        for qb in range(Q_PER_TILE // L):
            gi = (qb * L + lane) * NSAMPLE + s
            ti_v[s, pl.ds(qb * L, L)] = plsc.load_gather(idx_v, [gi])
    pltpu.sync_copy(ti_v, idx_hbm.at[b, :, qsl])
    _emit(gx_v, gxyz_hbm.at[b, 0, :, qsl])
    _emit(gy_v, gxyz_hbm.at[b, 1, :, qsl])
    _emit(gz_v, gxyz_hbm.at[b, 2, :, qsl])


def _gather_body(feat_hbm, idx_hbm, out_hbm, idxt_v,
                 row0_v, row1_v, out0_v, out1_v, rsem, osem):
    wid = _wid()
    b = wid // TILES_PER_BATCH
    cbase = (wid % TILES_PER_BATCH) * C_PER_TILE
    rows = [row0_v, row1_v]
    outs = [out0_v, out1_v]

    def _row_copy(c, buf):
        src = feat_hbm.at[pl.ds((b * C + cbase + c) * N, N)]
        return pltpu.async_copy(src, buf, rsem)

    def _out_copy(c, qc, buf):
        dst = out_hbm.at[b, cbase + c, :, pl.ds(qc * QCHUNK, QCHUNK)]
        return pltpu.async_copy(buf, dst, osem)

    def qc_body(qc, _):
        pltpu.sync_copy(
            idx_hbm.at[b, :, pl.ds(qc * QCHUNK, QCHUNK)], idxt_v)
        rd = {0: _row_copy(0, rows[0])}
        od = {}
        for c in range(C_PER_TILE):
            rd[c].wait()
            if c + 1 < C_PER_TILE:
                rd[c + 1] = _row_copy(c + 1, rows[(c + 1) % 2])
            if c - 2 in od:
                od[c - 2].wait()
            row_buf = rows[c % 2]
            out_buf = outs[c % 2]

            @plsc.parallel_loop(0, NSAMPLE * 2, 1, unroll=2)
            def gloop(o):
                s = o // 2
                qh = (o % 2) * (QCHUNK // 2)
                for qb in range(QCHUNK // L // 2):
                    idxv = idxt_v[s, pl.ds(qh + qb * L, L)]
                    out_buf[s, pl.ds(qh + qb * L, L)] = plsc.load_gather(
                        row_buf, [idxv])
            od[c] = _out_copy(c, qc, out_buf)
        od[C_PER_TILE - 2].wait()
        od[C_PER_TILE - 1].wait()
        return 0

    lax.fori_loop(0, NQ // QCHUNK, qc_body, 0)


@jax.jit
def _run(coords, q_flat, key_features):
    select = pl.kernel(
        _select_body,
        out_type=[
            jax.ShapeDtypeStruct((B, NSAMPLE, NQ), jnp.int32),
            jax.ShapeDtypeStruct((B, NSAMPLE, NQ), jnp.int32),
            jax.ShapeDtypeStruct((B, 3, NSAMPLE, NQ), jnp.float32),
        ],
        mesh=_mesh(),
        compiler_params=_params(),
        scratch_types=[
            pltpu.VMEM((N * 3,), jnp.float32),
            pltpu.VMEM((N,), jnp.float32),
            pltpu.VMEM((N,), jnp.float32),
            pltpu.VMEM((N,), jnp.float32),
            pltpu.VMEM((Q_PER_TILE * 6,), jnp.float32),
            pltpu.VMEM((Q_PER_TILE * NSAMPLE + NSAMPLE,), jnp.int32),
            pltpu.VMEM((Q_PER_TILE * NSAMPLE,), jnp.int32),
            pltpu.VMEM((Q_PER_TILE * NSAMPLE,), jnp.float32),
            pltpu.VMEM((Q_PER_TILE * NSAMPLE,), jnp.float32),
            pltpu.VMEM((Q_PER_TILE * NSAMPLE,), jnp.float32),
            pltpu.VMEM((NSAMPLE, Q_PER_TILE), jnp.float32),
            pltpu.VMEM((NSAMPLE, Q_PER_TILE), jnp.int32),
        ],
    )
    idx, mask_i, gxyz = select(coords, q_flat)

    gather = pl.kernel(
        _gather_body,
        out_type=jax.ShapeDtypeStruct((B, C, NSAMPLE, NQ), jnp.float32),
        mesh=_mesh(),
        compiler_params=_params(),
        scratch_types=[
            pltpu.VMEM((NSAMPLE, QCHUNK), jnp.int32),
            pltpu.VMEM((N,), jnp.float32),
            pltpu.VMEM((N,), jnp.float32),
            pltpu.VMEM((NSAMPLE, QCHUNK), jnp.float32),
            pltpu.VMEM((NSAMPLE, QCHUNK), jnp.float32),
            pltpu.SemaphoreType.DMA,
            pltpu.SemaphoreType.DMA,
        ],
    )
    gfeat = gather(key_features, idx)

    gxyz = jnp.transpose(gxyz, (0, 1, 3, 2))
    gfeat = jnp.transpose(gfeat, (0, 1, 3, 2))
    mask = jnp.transpose(mask_i, (0, 2, 1)).astype(bool)
    return gxyz, gfeat, mask


def kernel(key_xyz, key_features, query_xyz):
    return _run(key_xyz.reshape(-1), query_xyz.reshape(-1),
                key_features.reshape(-1))


# speedup vs baseline: 52.0161x; 1.1215x over previous
import jax
import jax.numpy as jnp
from jax import lax
from jax.experimental import pallas as pl
from jax.experimental.pallas import tpu as pltpu
from jax.experimental.pallas import tpu_sc as plsc

NSAMPLE = 64
L = 16
NUM_TILES = 32
B, N, NQ, C = 4, 8192, 1024, 128
Q_PER_TILE = NQ * B // NUM_TILES
TILES_PER_BATCH = NUM_TILES // B
C_PER_TILE = C // TILES_PER_BATCH
NKV = N // L
U = 16
QCHUNK = 512
CLEN = QCHUNK * NSAMPLE


def _mesh():
    return plsc.VectorSubcoreMesh(core_axis_name="c", subcore_axis_name="s")


def _params():
    return pltpu.CompilerParams(needs_layout_passes=False)


def _wid():
    return lax.axis_index("s") * 2 + lax.axis_index("c")


def _select_body(coords_hbm, query_hbm, idx_hbm, mask_hbm, gxyz_hbm,
                 cint_v, xs_v, ys_v, zs_v, q_v, idx_v, mask_v,
                 gx_v, gy_v, gz_v, t_v, ti_v):
    wid = _wid()
    b = wid // TILES_PER_BATCH
    qbase = (wid % TILES_PER_BATCH) * Q_PER_TILE

    pltpu.sync_copy(coords_hbm.at[pl.ds(b * N * 3, N * 3)], cint_v)
    pltpu.sync_copy(
        query_hbm.at[pl.ds((b * NQ + qbase) * 6, Q_PER_TILE * 6)], q_v)

    lane = jnp.arange(L, dtype=jnp.int32)
    zeros_i = jnp.zeros((L,), jnp.int32)

    @plsc.parallel_loop(0, N, L, unroll=8)
    def dloop(kb):
        idx3 = (kb + lane) * 3
        xs_v[pl.ds(kb, L)] = plsc.load_gather(cint_v, [idx3])
        ys_v[pl.ds(kb, L)] = plsc.load_gather(cint_v, [idx3 + 1])
        zs_v[pl.ds(kb, L)] = plsc.load_gather(cint_v, [idx3 + 2])

    def qloop(q, _):
        qsplat = jnp.full((L,), q, jnp.int32)
        q6 = qsplat * 6

        def qval(d):
            return plsc.load_gather(q_v, [q6 + d])

        cx, cy, cz = qval(0), qval(1), qval(2)
        hx, hy, hz = 0.5 * qval(3), 0.5 * qval(4), 0.5 * qval(5)
        obase = q * NSAMPLE

        for j in range(NSAMPLE // L):
            idx_v[pl.ds(obase + j * L, L)] = zeros_i

        def cond(carry):
            i, cnt = carry
            return jnp.logical_and(i < NKV, cnt < NSAMPLE)

        def body(carry):
            i, cnt = carry
            kb = i * L
            insides = []
            pcs = []
            for u in range(U):
                xv = xs_v[pl.ds(kb + u * L, L)]
                yv = ys_v[pl.ds(kb + u * L, L)]
                zv = zs_v[pl.ds(kb + u * L, L)]
                inside = jnp.logical_and(
                    jnp.logical_and(jnp.abs(xv - cx) <= hx,
                                    jnp.abs(yv - cy) <= hy),
                    jnp.abs(zv - cz) <= hz)
                insides.append(inside)
                pcs.append(plsc.all_reduce_population_count(inside))
            acc = pcs
            while len(acc) > 1:
                acc = [a + b for a, b in zip(acc[::2], acc[1::2])]
            tot_v = acc[0]
            tot = tot_v[0]

            @pl.when(tot > 0)
            def _():
                off = obase + cnt
                for u in range(U):
                    plsc.store_compressed(idx_v.at[pl.ds(off, L)],
                                          kb + u * L + lane, mask=insides[u])
                    if u + 1 < U:
                        off = off + pcs[u][0]

            return i + U, cnt + tot

        _, cnt = lax.while_loop(cond, body, (jnp.int32(0), jnp.int32(0)))
        cntv = jnp.full((L,), jnp.minimum(cnt, NSAMPLE), jnp.int32)

        for j in range(NSAMPLE // L):
            s_ids = j * L + lane
            idxv = idx_v[pl.ds(obase + j * L, L)]
            invalid = s_ids >= cntv
            if j == 0:
                invalid = jnp.logical_and(invalid, s_ids != 0)
            mask_v[pl.ds(obase + j * L, L)] = invalid.astype(jnp.int32)
            gx_v[pl.ds(obase + j * L, L)] = plsc.load_gather(xs_v, [idxv]) - cx
            gy_v[pl.ds(obase + j * L, L)] = plsc.load_gather(ys_v, [idxv]) - cy
            gz_v[pl.ds(obase + j * L, L)] = plsc.load_gather(zs_v, [idxv]) - cz
        return 0

    lax.fori_loop(0, Q_PER_TILE, qloop, 0)

    qsl = pl.ds(qbase, Q_PER_TILE)

    def _emit(src_v, dst):
        @plsc.parallel_loop(0, NSAMPLE, 1, unroll=2)
        def tloop(s):
            for qb in range(Q_PER_TILE // L):
                gi = (qb * L + lane) * NSAMPLE + s
                t_v[s, pl.ds(qb * L, L)] = plsc.load_gather(src_v, [gi])
        pltpu.sync_copy(t_v, dst)

    @plsc.parallel_loop(0, NSAMPLE, 1, unroll=2)
    def mloop(s):
        for qb in range(Q_PER_TILE // L):
            gi = (qb * L + lane) * NSAMPLE + s
            ti_v[s, pl.ds(qb * L, L)] = plsc.load_gather(mask_v, [gi])
    pltpu.sync_copy(ti_v, mask_hbm.at[b, :, qsl])

    @plsc.parallel_loop(0, NSAMPLE, 1, unroll=2)
    def iloop(s):
        for qb in range(Q_PER_TILE // L):
            gi = (qb * L + lane) * NSAMPLE + s
            ti_v[s, pl.ds(qb * L, L)] = plsc.load_gather(idx_v, [gi])
    pltpu.sync_copy(ti_v, idx_hbm.at[b, :, qsl])
    _emit(gx_v, gxyz_hbm.at[b, 0, :, qsl])
    _emit(gy_v, gxyz_hbm.at[b, 1, :, qsl])
    _emit(gz_v, gxyz_hbm.at[b, 2, :, qsl])


def _gather_body(feat_hbm, idx_hbm, out_hbm, idxt_v,
                 row0_v, row1_v, out0_v, out1_v, rsem, osem):
    wid = _wid()
    b = wid // TILES_PER_BATCH
    cbase = (wid % TILES_PER_BATCH) * C_PER_TILE
    rows = [row0_v, row1_v]
    outs = [out0_v, out1_v]

    def _row_copy(c, buf):
        src = feat_hbm.at[pl.ds((b * C + cbase + c) * N, N)]
        return pltpu.async_copy(src, buf, rsem)

    def _out_copy(c, qc, buf):
        dst = out_hbm.at[b, cbase + c, :, pl.ds(qc * QCHUNK, QCHUNK)]
        return pltpu.async_copy(buf, dst, osem)

    def qc_body(qc, _):
        pltpu.sync_copy(
            idx_hbm.at[b, :, pl.ds(qc * QCHUNK, QCHUNK)], idxt_v)
        rd = {0: _row_copy(0, rows[0])}
        od = {}
        for c in range(C_PER_TILE):
            rd[c].wait()
            if c + 1 < C_PER_TILE:
                rd[c + 1] = _row_copy(c + 1, rows[(c + 1) % 2])
            if c - 2 in od:
                od[c - 2].wait()
            row_buf = rows[c % 2]
            out_buf = outs[c % 2]

            @plsc.parallel_loop(0, NSAMPLE * 2, 1, unroll=2)
            def gloop(o):
                s = o // 2
                qh = (o % 2) * (QCHUNK // 2)
                for qb in range(QCHUNK // L // 2):
                    idxv = idxt_v[s, pl.ds(qh + qb * L, L)]
                    out_buf[s, pl.ds(qh + qb * L, L)] = plsc.load_gather(
                        row_buf, [idxv])
            od[c] = _out_copy(c, qc, out_buf)
        od[C_PER_TILE - 2].wait()
        od[C_PER_TILE - 1].wait()
        return 0

    lax.fori_loop(0, NQ // QCHUNK, qc_body, 0)


@jax.jit
def _run(coords, q_flat, key_features):
    select = pl.kernel(
        _select_body,
        out_type=[
            jax.ShapeDtypeStruct((B, NSAMPLE, NQ), jnp.int32),
            jax.ShapeDtypeStruct((B, NSAMPLE, NQ), jnp.int32),
            jax.ShapeDtypeStruct((B, 3, NSAMPLE, NQ), jnp.float32),
        ],
        mesh=_mesh(),
        compiler_params=_params(),
        scratch_types=[
            pltpu.VMEM((N * 3,), jnp.float32),
            pltpu.VMEM((N,), jnp.float32),
            pltpu.VMEM((N,), jnp.float32),
            pltpu.VMEM((N,), jnp.float32),
            pltpu.VMEM((Q_PER_TILE * 6,), jnp.float32),
            pltpu.VMEM((Q_PER_TILE * NSAMPLE + NSAMPLE,), jnp.int32),
            pltpu.VMEM((Q_PER_TILE * NSAMPLE,), jnp.int32),
            pltpu.VMEM((Q_PER_TILE * NSAMPLE,), jnp.float32),
            pltpu.VMEM((Q_PER_TILE * NSAMPLE,), jnp.float32),
            pltpu.VMEM((Q_PER_TILE * NSAMPLE,), jnp.float32),
            pltpu.VMEM((NSAMPLE, Q_PER_TILE), jnp.float32),
            pltpu.VMEM((NSAMPLE, Q_PER_TILE), jnp.int32),
        ],
    )
    idx, mask_i, gxyz = select(coords, q_flat)

    gather = pl.kernel(
        _gather_body,
        out_type=jax.ShapeDtypeStruct((B, C, NSAMPLE, NQ), jnp.float32),
        mesh=_mesh(),
        compiler_params=_params(),
        scratch_types=[
            pltpu.VMEM((NSAMPLE, QCHUNK), jnp.int32),
            pltpu.VMEM((N,), jnp.float32),
            pltpu.VMEM((N,), jnp.float32),
            pltpu.VMEM((NSAMPLE, QCHUNK), jnp.float32),
            pltpu.VMEM((NSAMPLE, QCHUNK), jnp.float32),
            pltpu.SemaphoreType.DMA,
            pltpu.SemaphoreType.DMA,
        ],
    )
    gfeat = gather(key_features, idx)

    gxyz = jnp.transpose(gxyz, (0, 1, 3, 2))
    gfeat = jnp.transpose(gfeat, (0, 1, 3, 2))
    mask = jnp.transpose(mask_i, (0, 2, 1)).astype(bool)
    return gxyz, gfeat, mask


def kernel(key_xyz, key_features, query_xyz):
    return _run(key_xyz.reshape(-1), query_xyz.reshape(-1),
                key_features.reshape(-1))
